# baseline (speedup 1.0000x reference)
import functools

import jax
import jax.numpy as jnp
from jax import lax
from jax.experimental import pallas as pl
from jax.experimental.pallas import tpu as pltpu
from jax.experimental.pallas import tpu_sc as plsc

VOCAB = 100000
SEQ_LEN = 200
EMBED_DIM = 128
BATCH = 1024

SPLITS = ((0, 128), (128, 72))
LANES = 16
NBUF = 4
NUM_WORKERS = 32
NUM_CORES = 2
BATCH_PER_W = BATCH // NUM_WORKERS


def _sc_body(idx_hbm, table_hbm, pos_hbm, out_hbm, pos_v,
             bufs, idxbs, gsems, ssems, isems):
    wid = lax.axis_index("s") * NUM_CORES + lax.axis_index("c")
    b_base = wid * BATCH_PER_W

    pltpu.sync_copy(pos_hbm, pos_v)

    def idx_copy(seq, b):
        return pltpu.make_async_copy(
            idx_hbm.at[pl.ds((b_base + seq) * SEQ_LEN, SEQ_LEN)], idxbs[b],
            isems[b])

    def gather_half(b, h):
        off, n = SPLITS[h]
        return (table_hbm.at[idxbs[b].at[pl.ds(off, n)]],
                bufs[b].at[pl.ds(off, n), :],
                gsems[b])

    def start_gathers(b):
        for h in range(len(SPLITS)):
            src, dst, sem = gather_half(b, h)
            pltpu.async_copy(src, dst, sem)

    def wait_gather_half(b, h):
        src, dst, sem = gather_half(b, h)
        pltpu.make_async_copy(src, dst, sem).wait()

    def add_pos_half(b, h):
        off, n = SPLITS[h]
        buf = bufs[b]

        @pl.loop(off, off + n)
        def _rows(r):
            for j in range(EMBED_DIM // LANES):
                sl = pl.ds(j * LANES, LANES)
                plsc.addupdate(buf.at[r, sl], pos_v[r, sl])

    def store_half(seq, b, h):
        off, n = SPLITS[h]
        return pltpu.make_async_copy(
            bufs[b].at[pl.ds(off, n), :],
            out_hbm.at[b_base + seq, pl.ds(off, n), :], ssems[b])

    def wait_store(seq, b):
        for h in range(len(SPLITS)):
            store_half(seq, b, h).wait()

    for s in range(NBUF - 1):
        idx_copy(s, s).start()
        idx_copy(s, s).wait()
        start_gathers(s)
    idx_copy(NBUF - 1, NBUF - 1).start()

    @pl.loop(0, BATCH_PER_W, step=NBUF)
    def _ring(c):
        for b in range(NBUF):
            seq = c + b
            nb = (b + NBUF - 1) % NBUF
            wait_gather_half(b, 0)

            @pl.when(seq > 0)
            def _():
                wait_store(seq - 1, nb)

            @pl.when(seq < BATCH_PER_W - NBUF + 1)
            def _():
                idx_copy(seq + NBUF - 1, nb).wait()
                start_gathers(nb)

            add_pos_half(b, 0)
            store_half(seq, b, 0).start()
            wait_gather_half(b, 1)
            add_pos_half(b, 1)
            store_half(seq, b, 1).start()

            @pl.when(seq < BATCH_PER_W - NBUF)
            def _():
                idx_copy(seq + NBUF, b).start()

    wait_store(BATCH_PER_W - 1, (BATCH_PER_W - 1) % NBUF)


@jax.jit
def _embed(idx2d, token_table, pos_table):
    mesh = plsc.VectorSubcoreMesh(core_axis_name="c", subcore_axis_name="s")

    def body(idx_hbm, table_hbm, pos_hbm, out_hbm, pos_v,
             b0, b1, b2, b3, i0, i1, i2, i3,
             g0, g1, g2, g3, s0, s1, s2, s3, m0, m1, m2, m3):
        _sc_body(idx_hbm, table_hbm, pos_hbm, out_hbm, pos_v,
                 (b0, b1, b2, b3), (i0, i1, i2, i3),
                 (g0, g1, g2, g3), (s0, s1, s2, s3), (m0, m1, m2, m3))

    f = functools.partial(
        pl.kernel,
        out_type=jax.ShapeDtypeStruct((BATCH, SEQ_LEN, EMBED_DIM), jnp.float32),
        mesh=mesh,
        scratch_types=(
            [pltpu.VMEM((SEQ_LEN, EMBED_DIM), jnp.float32)]
            + [pltpu.VMEM((SEQ_LEN, EMBED_DIM), jnp.float32)] * NBUF
            + [pltpu.VMEM((SEQ_LEN,), jnp.int32)] * NBUF
            + [pltpu.SemaphoreType.DMA] * (3 * NBUF)
        ),
    )(body)
    return f(idx2d, token_table, pos_table)


def kernel(inputs, token_table, pos_table):
    return _embed(inputs.reshape(-1).astype(jnp.int32), token_table,
                  pos_table.astype(jnp.float32))


# speedup vs baseline: 1.0963x; 1.0963x over previous
import functools

import jax
import jax.numpy as jnp
from jax import lax
from jax.experimental import pallas as pl
from jax.experimental.pallas import tpu as pltpu
from jax.experimental.pallas import tpu_sc as plsc

VOCAB = 100000
SEQ_LEN = 200
EMBED_DIM = 128
BATCH = 1024

SPLITS = ((0, 128), (128, 72))
LANES = 16
NBUF = 3
NUM_WORKERS = 32
NUM_CORES = 2
BATCH_PER_W = BATCH // NUM_WORKERS
ROWS_PER_W = BATCH_PER_W * SEQ_LEN
STEADY = BATCH_PER_W - 2


def _sc_body(idx_hbm, table_hbm, pos_hbm, out_hbm, idx_v, pos_v,
             buf0, buf1, buf2, gsem0, gsem1, gsem2, ssem0, ssem1, ssem2):
    wid = lax.axis_index("s") * NUM_CORES + lax.axis_index("c")
    b_base = wid * BATCH_PER_W

    pltpu.sync_copy(idx_hbm.at[pl.ds(b_base, BATCH_PER_W), :], idx_v)
    pltpu.sync_copy(pos_hbm, pos_v)

    bufs = (buf0, buf1, buf2)
    gsems = (gsem0, gsem1, gsem2)
    ssems = (ssem0, ssem1, ssem2)

    def gather_half(seq, b, h):
        off, n = SPLITS[h]
        return (table_hbm.at[idx_v.at[seq, pl.ds(off, n)]],
                bufs[b].at[pl.ds(off, n), :],
                gsems[b])

    def start_gathers(seq, b):
        for h in range(len(SPLITS)):
            src, dst, sem = gather_half(seq, b, h)
            pltpu.async_copy(src, dst, sem)

    def wait_gather_half(seq, b, h):
        src, dst, sem = gather_half(seq, b, h)
        pltpu.make_async_copy(src, dst, sem).wait()

    def add_pos_half(b, h):
        off, n = SPLITS[h]
        buf = bufs[b]

        @pl.loop(off, off + n)
        def _rows(r):
            for j in range(EMBED_DIM // LANES):
                sl = pl.ds(j * LANES, LANES)
                plsc.addupdate(buf.at[r, sl], pos_v[r, sl])

    def start_store_half(seq, b, h):
        off, n = SPLITS[h]
        pltpu.async_copy(bufs[b].at[pl.ds(off, n), :],
                         out_hbm.at[b_base + seq, pl.ds(off, n), :], ssems[b])

    def wait_store(seq, b):
        for h in range(len(SPLITS)):
            off, n = SPLITS[h]
            pltpu.make_async_copy(bufs[b].at[pl.ds(off, n), :],
                                  out_hbm.at[b_base + seq, pl.ds(off, n), :],
                                  ssems[b]).wait()

    def slot(seq, b, nb, refill):
        if refill is not None:
            refill()
        wait_gather_half(seq, b, 0)
        add_pos_half(b, 0)
        start_store_half(seq, b, 0)
        wait_gather_half(seq, b, 1)
        add_pos_half(b, 1)
        start_store_half(seq, b, 1)

    for b in range(2):
        start_gathers(b, b)

    @pl.loop(0, STEADY, step=NBUF)
    def _ring(c):
        for b in range(NBUF):
            seq = c + b
            nb = (b + 2) % NBUF

            def refill(seq=seq, nb=nb):
                @pl.when(seq > 0)
                def _():
                    wait_store(seq - 1, nb)
                start_gathers(seq + 2, nb)

            slot(seq, b, nb, refill)

    for seq in (BATCH_PER_W - 2, BATCH_PER_W - 1):
        b = seq % NBUF
        nb = (b + 2) % NBUF

        def refill(seq=seq, nb=nb):
            wait_store(seq - 1, nb)

        slot(seq, b, nb, refill)

    wait_store(BATCH_PER_W - 1, (BATCH_PER_W - 1) % NBUF)


@jax.jit
def _embed(idx_flat, token_table, pos_table):
    mesh = plsc.VectorSubcoreMesh(core_axis_name="c", subcore_axis_name="s")
    f = functools.partial(
        pl.kernel,
        out_type=jax.ShapeDtypeStruct((BATCH, SEQ_LEN, EMBED_DIM), jnp.float32),
        mesh=mesh,
        scratch_types=[
            pltpu.VMEM((BATCH_PER_W, SEQ_LEN), jnp.int32),
            pltpu.VMEM((SEQ_LEN, EMBED_DIM), jnp.float32),
            pltpu.VMEM((SEQ_LEN, EMBED_DIM), jnp.float32),
            pltpu.VMEM((SEQ_LEN, EMBED_DIM), jnp.float32),
            pltpu.VMEM((SEQ_LEN, EMBED_DIM), jnp.float32),
            pltpu.SemaphoreType.DMA,
            pltpu.SemaphoreType.DMA,
            pltpu.SemaphoreType.DMA,
            pltpu.SemaphoreType.DMA,
            pltpu.SemaphoreType.DMA,
            pltpu.SemaphoreType.DMA,
        ],
    )(_sc_body)
    return f(idx_flat, token_table, pos_table)


def kernel(inputs, token_table, pos_table):
    return _embed(inputs.astype(jnp.int32), token_table,
                  pos_table.astype(jnp.float32))


# speedup vs baseline: 1.1094x; 1.0119x over previous
import functools

import jax
import jax.numpy as jnp
from jax import lax
from jax.experimental import pallas as pl
from jax.experimental.pallas import tpu as pltpu
from jax.experimental.pallas import tpu_sc as plsc

VOCAB = 100000
SEQ_LEN = 200
EMBED_DIM = 128
BATCH = 1024

SPLITS = ((0, 128), (128, 72))
LANES = 16
NBUF = 3
NUM_WORKERS = 32
NUM_CORES = 2
BATCH_PER_W = BATCH // NUM_WORKERS
ROWS_PER_W = BATCH_PER_W * SEQ_LEN
STEADY = BATCH_PER_W - 2


def _sc_body(idx_hbm, table_hbm, pos_hbm, out_hbm, idx_v, pos_v,
             buf0, buf1, buf2, gsem0, gsem1, gsem2, ssem0, ssem1, ssem2):
    wid = lax.axis_index("s") * NUM_CORES + lax.axis_index("c")
    b_base = wid * BATCH_PER_W

    pltpu.sync_copy(idx_hbm.at[pl.ds(b_base, BATCH_PER_W), :], idx_v)
    pltpu.sync_copy(pos_hbm, pos_v)

    bufs = (buf0, buf1, buf2)
    gsems = (gsem0, gsem1, gsem2)
    ssems = (ssem0, ssem1, ssem2)

    def gather_half(seq, b, h):
        off, n = SPLITS[h]
        return (table_hbm.at[idx_v.at[seq, pl.ds(off, n)]],
                bufs[b].at[pl.ds(off, n), :],
                gsems[b])

    def start_gathers(seq, b):
        for h in range(len(SPLITS)):
            src, dst, sem = gather_half(seq, b, h)
            pltpu.async_copy(src, dst, sem)

    def wait_gather_half(seq, b, h):
        src, dst, sem = gather_half(seq, b, h)
        pltpu.make_async_copy(src, dst, sem).wait()

    def add_pos_half(b, h):
        off, n = SPLITS[h]
        buf = bufs[b]

        @pl.loop(off, off + n)
        def _rows(r):
            for j in range(EMBED_DIM // LANES):
                sl = pl.ds(j * LANES, LANES)
                plsc.addupdate(buf.at[r, sl], pos_v[r, sl])

    def start_store_half(seq, b, h):
        off, n = SPLITS[h]
        pltpu.async_copy(bufs[b].at[pl.ds(off, n), :],
                         out_hbm.at[b_base + seq, pl.ds(off, n), :], ssems[b])

    def wait_store(seq, b):
        for h in range(len(SPLITS)):
            off, n = SPLITS[h]
            pltpu.make_async_copy(bufs[b].at[pl.ds(off, n), :],
                                  out_hbm.at[b_base + seq, pl.ds(off, n), :],
                                  ssems[b]).wait()

    def slot(seq, b, nb, refill):
        wait_gather_half(seq, b, 0)
        add_pos_half(b, 0)
        start_store_half(seq, b, 0)
        if refill is not None:
            refill()
        wait_gather_half(seq, b, 1)
        add_pos_half(b, 1)
        start_store_half(seq, b, 1)

    for b in range(2):
        start_gathers(b, b)

    @pl.loop(0, STEADY, step=NBUF)
    def _ring(c):
        for b in range(NBUF):
            seq = c + b
            nb = (b + 2) % NBUF

            def refill(seq=seq, nb=nb):
                @pl.when(seq > 0)
                def _():
                    wait_store(seq - 1, nb)
                start_gathers(seq + 2, nb)

            slot(seq, b, nb, refill)

    for seq in (BATCH_PER_W - 2, BATCH_PER_W - 1):
        b = seq % NBUF
        nb = (b + 2) % NBUF

        def refill(seq=seq, nb=nb):
            wait_store(seq - 1, nb)

        slot(seq, b, nb, refill)

    wait_store(BATCH_PER_W - 1, (BATCH_PER_W - 1) % NBUF)


@jax.jit
def _embed(idx_flat, token_table, pos_table):
    mesh = plsc.VectorSubcoreMesh(core_axis_name="c", subcore_axis_name="s")
    f = functools.partial(
        pl.kernel,
        out_type=jax.ShapeDtypeStruct((BATCH, SEQ_LEN, EMBED_DIM), jnp.float32),
        mesh=mesh,
        scratch_types=[
            pltpu.VMEM((BATCH_PER_W, SEQ_LEN), jnp.int32),
            pltpu.VMEM((SEQ_LEN, EMBED_DIM), jnp.float32),
            pltpu.VMEM((SEQ_LEN, EMBED_DIM), jnp.float32),
            pltpu.VMEM((SEQ_LEN, EMBED_DIM), jnp.float32),
            pltpu.VMEM((SEQ_LEN, EMBED_DIM), jnp.float32),
            pltpu.SemaphoreType.DMA,
            pltpu.SemaphoreType.DMA,
            pltpu.SemaphoreType.DMA,
            pltpu.SemaphoreType.DMA,
            pltpu.SemaphoreType.DMA,
            pltpu.SemaphoreType.DMA,
        ],
    )(_sc_body)
    return f(idx_flat, token_table, pos_table)


def kernel(inputs, token_table, pos_table):
    return _embed(inputs.astype(jnp.int32), token_table,
                  pos_table.astype(jnp.float32))
